# baseline (speedup 1.0000x reference)
import functools

import jax
import jax.numpy as jnp
import numpy as np
from jax.experimental import pallas as pl
from jax.experimental.pallas import tpu as pltpu

_DIN = 48
_DOUT = 192
_DT = 16
_NPLANES = _DT // 4 + 2
_NBUF = 8


def _interp_matrix(n_in: int, n_out: int) -> np.ndarray:
    m = np.zeros((n_in, n_out), dtype=np.float32)
    scale = n_in / n_out
    for o in range(n_out):
        c = (o + 0.5) * scale - 0.5
        i0 = int(np.floor(c))
        t = c - i0
        m[min(max(i0, 0), n_in - 1), o] += 1.0 - t
        m[min(max(i0 + 1, 0), n_in - 1), o] += t
    return m


def _body(a_ref, mht_ref, mw_ref, o_hbm, scratch, sems):
    i = pl.program_id(0)
    nsteps = pl.num_programs(0)
    mht = mht_ref[...]
    mw = mw_ref[...]
    d0 = i * (_DT // 4) - 1
    u = []
    for c in range(3):
        uc = []
        for j in range(_NPLANES):
            dj = jnp.clip(d0 + j, 0, _DIN - 1)
            s2 = jnp.dot(mht, a_ref[c, dj], preferred_element_type=jnp.float32)
            uc.append(jnp.dot(s2, mw, preferred_element_type=jnp.float32))
        u.append(uc)
    planes_per_step = 3 * _DT
    for k in range(_DT):
        i0rel = (2 * k - 3) // 8
        frac = k * 0.25 - 0.375 - i0rel
        j0 = i0rel + 1
        od = i * _DT + k
        for c in range(3):
            p = k * 3 + c
            gidx = i * planes_per_step + p
            slot = jax.lax.rem(gidx, _NBUF)
            cp = pltpu.make_async_copy(
                scratch.at[slot], o_hbm.at[c, od], sems.at[slot]
            )
            @pl.when(gidx >= _NBUF)
            def _():
                cp.wait()

            scratch[slot] = (1.0 - frac) * u[c][j0] + frac * u[c][j0 + 1]
            cp.start()
    @pl.when(i == nsteps - 1)
    def _():
        for s in range(_NBUF):
            pltpu.make_async_copy(
                scratch.at[s], o_hbm.at[0, 0], sems.at[s]
            ).wait()


@jax.jit
def _upsample(disp):
    a = jnp.transpose(jnp.reshape(disp, (_DIN, _DIN, _DIN, 3)), (3, 0, 1, 2))
    mw = jnp.asarray(_interp_matrix(_DIN, _DOUT))
    mht = mw.T
    out = pl.pallas_call(
        _body,
        grid=(_DOUT // _DT,),
        in_specs=[
            pl.BlockSpec((3, _DIN, _DIN, _DIN), lambda i: (0, 0, 0, 0)),
            pl.BlockSpec((_DOUT, _DIN), lambda i: (0, 0)),
            pl.BlockSpec((_DIN, _DOUT), lambda i: (0, 0)),
        ],
        out_specs=pl.BlockSpec(memory_space=pl.ANY),
        out_shape=jax.ShapeDtypeStruct((3, _DOUT, _DOUT, _DOUT), jnp.float32),
        scratch_shapes=[
            pltpu.VMEM((_NBUF, _DOUT, _DOUT), jnp.float32),
            pltpu.SemaphoreType.DMA((_NBUF,)),
        ],
    )(a, mht, mw)
    return jnp.reshape(out, (1, 3, _DOUT, _DOUT, _DOUT))


def kernel(kpts, disp, features_fixed):
    del kpts, features_fixed
    return _upsample(disp)


# speedup vs baseline: 1.7395x; 1.7395x over previous
import functools

import jax
import jax.numpy as jnp
import numpy as np
from jax.experimental import pallas as pl
from jax.experimental.pallas import tpu as pltpu

_DIN = 48
_DOUT = 192
_DT = 16
_NPLANES = _DT // 4 + 2
_NBUF = 6
_GP = 8


def _interp_matrix(n_in: int, n_out: int) -> np.ndarray:
    m = np.zeros((n_in, n_out), dtype=np.float32)
    scale = n_in / n_out
    for o in range(n_out):
        c = (o + 0.5) * scale - 0.5
        i0 = int(np.floor(c))
        t = c - i0
        m[min(max(i0, 0), n_in - 1), o] += 1.0 - t
        m[min(max(i0 + 1, 0), n_in - 1), o] += t
    return m


def _body(a_ref, mht_ref, mw_ref, o_hbm, scratch, sems):
    i = pl.program_id(0)
    nsteps = pl.num_programs(0)
    mht = mht_ref[...]
    mw = mw_ref[...]
    d0 = i * (_DT // 4) - 1
    u = []
    for c in range(3):
        uc = []
        for j in range(_NPLANES):
            dj = jnp.clip(d0 + j, 0, _DIN - 1)
            s2 = jnp.dot(mht, a_ref[c, dj], preferred_element_type=jnp.float32)
            uc.append(jnp.dot(s2, mw, preferred_element_type=jnp.float32))
        u.append(uc)
    ngroups = _DT // _GP
    chunks_per_step = 3 * ngroups
    for c in range(3):
        for g in range(ngroups):
            gidx = i * chunks_per_step + c * ngroups + g
            slot = jax.lax.rem(gidx, _NBUF)
            od0 = i * _DT + g * _GP
            cp = pltpu.make_async_copy(
                scratch.at[slot], o_hbm.at[c, pl.ds(od0, _GP)], sems.at[slot]
            )
            @pl.when(gidx >= _NBUF)
            def _():
                cp.wait()

            for k2 in range(_GP):
                k = g * _GP + k2
                i0rel = (2 * k - 3) // 8
                frac = k * 0.25 - 0.375 - i0rel
                j0 = i0rel + 1
                scratch[slot, k2] = (1.0 - frac) * u[c][j0] + frac * u[c][j0 + 1]
            cp.start()
    @pl.when(i == nsteps - 1)
    def _():
        for s in range(_NBUF):
            pltpu.make_async_copy(
                scratch.at[s], o_hbm.at[0, pl.ds(0, _GP)], sems.at[s]
            ).wait()


@jax.jit
def _upsample(disp):
    a = jnp.transpose(jnp.reshape(disp, (_DIN, _DIN, _DIN, 3)), (3, 0, 1, 2))
    mw = jnp.asarray(_interp_matrix(_DIN, _DOUT))
    mht = mw.T
    out = pl.pallas_call(
        _body,
        grid=(_DOUT // _DT,),
        in_specs=[
            pl.BlockSpec((3, _DIN, _DIN, _DIN), lambda i: (0, 0, 0, 0)),
            pl.BlockSpec((_DOUT, _DIN), lambda i: (0, 0)),
            pl.BlockSpec((_DIN, _DOUT), lambda i: (0, 0)),
        ],
        out_specs=pl.BlockSpec(memory_space=pl.ANY),
        out_shape=jax.ShapeDtypeStruct((3, _DOUT, _DOUT, _DOUT), jnp.float32),
        scratch_shapes=[
            pltpu.VMEM((_NBUF, _GP, _DOUT, _DOUT), jnp.float32),
            pltpu.SemaphoreType.DMA((_NBUF,)),
        ],
    )(a, mht, mw)
    return jnp.reshape(out, (1, 3, _DOUT, _DOUT, _DOUT))


def kernel(kpts, disp, features_fixed):
    del kpts, features_fixed
    return _upsample(disp)
